# Initial kernel scaffold; baseline (speedup 1.0000x reference)
import jax
import jax.numpy as jnp
from jax.experimental import pallas as pl


def kernel(X, edge_index, Y, idx, uS_c1_W, uS_c1_b, uS_mu_W, uS_mu_b, uS_ls_W, uS_ls_b, uY_c1_W, uY_c1_b, uY_mu_W, uY_mu_b, uY_ls_W, uY_ls_b, Xd1_W, Xd1_b, Xd2_W, Xd2_b, Yd1_W, Yd1_b, Yd2_W, Yd2_b, Sd1_W, Sd1_b, Sd2_W, Sd2_b):
    raise NotImplementedError("write your pallas kernel here")


# speedup vs baseline: 1.3004x; 1.3004x over previous
import functools

import jax
import jax.numpy as jnp
from jax import lax
from jax.experimental import pallas as pl
from jax.experimental.pallas import tpu as pltpu
from jax.experimental.pallas import tpu_sc as plsc

NN = 4096
HI = jax.lax.Precision.HIGHEST


def _mmscale_body(absx, x_ref, w_ref, dis_ref, o_ref):
    x = x_ref[...]
    if absx:
        x = jnp.abs(x)
    xw = jnp.dot(x, w_ref[...], preferred_element_type=jnp.float32)
    o_ref[...] = xw * dis_ref[...][:, None]


def _mmscale(x, w, dis, absx=False):
    n, _ = x.shape
    c = w.shape[1]
    return pl.pallas_call(
        functools.partial(_mmscale_body, absx),
        out_shape=jax.ShapeDtypeStruct((n, c), jnp.float32),
    )(x, w, dis)


def _bn(p):
    m = jnp.mean(p, axis=0)
    v = jnp.mean((p - m) ** 2, axis=0)
    return (p - m) / jnp.sqrt(v + 1e-5)


def _finish_body(mode, acc_ref, t_ref, dis_ref, b_ref, o_ref):
    acc = acc_ref[0] + acc_ref[1]
    p = (acc + t_ref[...]) * dis_ref[...][:, None] + b_ref[...][None, :]
    if mode == "bn_relu":
        o_ref[...] = jnp.maximum(_bn(p), 0.0)
    elif mode == "bn":
        o_ref[...] = _bn(p)
    else:
        o_ref[...] = p


def _finish(acc, t, dis, b, mode):
    return pl.pallas_call(
        functools.partial(_finish_body, mode),
        out_shape=jax.ShapeDtypeStruct(t.shape, jnp.float32),
    )(acc, t, dis, b)


def _reparam_body(mu_ref, lv_ref, eps_ref, u_ref):
    u_ref[...] = eps_ref[...] * jnp.exp(0.5 * lv_ref[...]) + mu_ref[...]


def _reparam(mu, lv, eps):
    return pl.pallas_call(
        _reparam_body,
        out_shape=jax.ShapeDtypeStruct(mu.shape, jnp.float32),
    )(mu, lv, eps)


def _head_body(yo_ref, so_ref, yn_ref, sn_ref, sl_ref):
    yo = yo_ref[...]
    so = so_ref[...]
    yn_ref[...] = jax.nn.softmax(yo, axis=-1)
    sn_ref[...] = jax.nn.softmax(so, axis=-1)
    sl_ref[...] = jax.nn.sigmoid(so)


def _heads(yo, so):
    n = yo.shape[0]
    s2 = jax.ShapeDtypeStruct((n, 2), jnp.float32)
    return pl.pallas_call(_head_body, out_shape=(s2, s2, s2))(yo, so)


_AR = 512


def _adj1_body(fb_ref, ff_ref, nz_ref, au_ref):
    i = pl.program_id(0)
    raw = lax.dot_general(fb_ref[...], ff_ref[...],
                          (((1,), (1,)), ((), ())),
                          preferred_element_type=jnp.float32)
    rows = i * _AR + lax.broadcasted_iota(jnp.int32, raw.shape, 0)
    cols = lax.broadcasted_iota(jnp.int32, raw.shape, 1)
    upper = jnp.where(cols > rows, raw, 0.0)
    g = -jnp.log(-jnp.log(nz_ref[...] + 1e-9) + 1e-9)
    logits = upper + g
    m = jnp.max(logits, axis=-1, keepdims=True)
    e = jnp.exp(logits - m)
    au_ref[...] = e / jnp.sum(e, axis=-1, keepdims=True)


def _adj1(feat, noise):
    grid = NN // _AR
    return pl.pallas_call(
        _adj1_body,
        grid=(grid,),
        in_specs=[
            pl.BlockSpec((_AR, feat.shape[1]), lambda i: (i, 0)),
            pl.BlockSpec((NN, feat.shape[1]), lambda i: (0, 0)),
            pl.BlockSpec((_AR, NN), lambda i: (i, 0)),
        ],
        out_specs=pl.BlockSpec((_AR, NN), lambda i: (i, 0)),
        out_shape=jax.ShapeDtypeStruct((NN, NN), jnp.float32),
    )(feat, feat, noise)


def _adj2_body(a_ref, at_ref, prob_ref, an_ref):
    p = a_ref[...] + at_ref[...].T
    prob_ref[...] = p
    an_ref[...] = jnp.where(p > 0.5, 1.0, 0.0)


def _adj2(au):
    grid = NN // _AR
    s = jax.ShapeDtypeStruct((NN, NN), jnp.float32)
    return pl.pallas_call(
        _adj2_body,
        grid=(grid, grid),
        in_specs=[
            pl.BlockSpec((_AR, _AR), lambda i, j: (i, j)),
            pl.BlockSpec((_AR, _AR), lambda i, j: (j, i)),
        ],
        out_specs=[pl.BlockSpec((_AR, _AR), lambda i, j: (i, j)),
                   pl.BlockSpec((_AR, _AR), lambda i, j: (i, j))],
        out_shape=(s, s),
    )(au, au)


_NC = 2
_NS = 16
_NW = _NC * _NS
_EK = 128


def _sc_segsum(xs, srci, dsti):
    n, c = xs.shape
    e = srci.shape[0]
    per_w = e // _NW
    nch = per_w // _EK
    slab = n // _NS
    zrow = jnp.zeros((16, c), jnp.float32)
    mesh = plsc.VectorSubcoreMesh(core_axis_name="c", subcore_axis_name="s")

    @functools.partial(
        pl.kernel, mesh=mesh,
        out_type=jax.ShapeDtypeStruct((_NC, n, c), jnp.float32),
        scratch_types=[
            pltpu.VMEM((_EK,), jnp.int32),
            pltpu.VMEM((1, _EK), jnp.int32),
            pltpu.VMEM((_EK, c), jnp.float32),
            pltpu.VMEM((16, c), jnp.float32),
            pltpu.VMEM_SHARED((n, c), jnp.float32),
            pltpu.SemaphoreType.DMA,
        ],
    )
    def k(xs_hbm, src_hbm, dst_hbm, z_hbm, out_hbm, sidx, didx, rows,
          zbuf, acc, sem):
        cid = lax.axis_index("c")
        sid = lax.axis_index("s")
        wid = sid * _NC + cid
        pltpu.sync_copy(z_hbm, zbuf)

        def zloop(i, carry):
            pltpu.sync_copy(zbuf, acc.at[pl.ds(sid * slab + i * 16, 16)])
            return carry

        lax.fori_loop(0, slab // 16, zloop, 0)
        plsc.subcore_barrier()
        base0 = wid * per_w

        def chunk(i, carry):
            b = base0 + i * _EK
            pltpu.sync_copy(src_hbm.at[pl.ds(b, _EK)], sidx)
            pltpu.sync_copy(dst_hbm.at[pl.ds(b, _EK)], didx.at[0])
            pltpu.async_copy(xs_hbm.at[sidx], rows, sem).wait()
            pltpu.sync_copy(rows, acc.at[didx.at[0]], add=True)
            return carry

        lax.fori_loop(0, nch, chunk, 0)
        plsc.subcore_barrier()
        pltpu.sync_copy(acc.at[pl.ds(sid * slab, slab)],
                        out_hbm.at[cid].at[pl.ds(sid * slab, slab)])

    return k(xs, srci, dsti, zrow)


def _sc_degree(dsti):
    n = NN
    e = dsti.shape[0]
    per_w = e // _NW
    nch = per_w // _EK
    slab = n // _NS
    ones = jnp.ones((_EK, 128), jnp.float32)
    zrow = jnp.zeros((16, 128), jnp.float32)
    mesh = plsc.VectorSubcoreMesh(core_axis_name="c", subcore_axis_name="s")

    @functools.partial(
        pl.kernel, mesh=mesh,
        out_type=jax.ShapeDtypeStruct((_NC, n, 128), jnp.float32),
        scratch_types=[
            pltpu.VMEM((1, _EK), jnp.int32),
            pltpu.VMEM((_EK, 128), jnp.float32),
            pltpu.VMEM((16, 128), jnp.float32),
            pltpu.VMEM_SHARED((n, 128), jnp.float32),
        ],
    )
    def k(dst_hbm, ones_hbm, z_hbm, out_hbm, didx, onev, zbuf, acc):
        cid = lax.axis_index("c")
        sid = lax.axis_index("s")
        wid = sid * _NC + cid
        pltpu.sync_copy(z_hbm, zbuf)
        pltpu.sync_copy(ones_hbm, onev)

        def zloop(i, carry):
            pltpu.sync_copy(zbuf, acc.at[pl.ds(sid * slab + i * 16, 16)])
            return carry

        lax.fori_loop(0, slab // 16, zloop, 0)
        plsc.subcore_barrier()
        base0 = wid * per_w

        def chunk(i, carry):
            b = base0 + i * _EK
            pltpu.sync_copy(dst_hbm.at[pl.ds(b, _EK)], didx.at[0])
            pltpu.sync_copy(onev, acc.at[didx.at[0]], add=True)
            return carry

        lax.fori_loop(0, nch, chunk, 0)
        plsc.subcore_barrier()
        pltpu.sync_copy(acc.at[pl.ds(sid * slab, slab)],
                        out_hbm.at[cid].at[pl.ds(sid * slab, slab)])

    return k(dsti, ones, zrow)


def _dis_body(d_ref, o_ref):
    d = d_ref[0, :, 0] + d_ref[1, :, 0] + 1.0
    o_ref[...] = d ** -0.5


def _dis_from_deg(degout):
    return pl.pallas_call(
        _dis_body,
        out_shape=jax.ShapeDtypeStruct((NN,), jnp.float32),
    )(degout)


def _pad128(w, b):
    c = w.shape[1]
    if c == 128:
        return w, b
    return (jnp.pad(w, ((0, 0), (0, 128 - c))), jnp.pad(b, (0, 128 - c)))


def kernel(X, edge_index, Y, idx, uS_c1_W, uS_c1_b, uS_mu_W, uS_mu_b,
           uS_ls_W, uS_ls_b, uY_c1_W, uY_c1_b, uY_mu_W, uY_mu_b, uY_ls_W,
           uY_ls_b, Xd1_W, Xd1_b, Xd2_W, Xd2_b, Yd1_W, Yd1_b, Yd2_W, Yd2_b,
           Sd1_W, Sd1_b, Sd2_W, Sd2_b):
    n = X.shape[0]
    src = edge_index[0]
    dst = edge_index[1]

    eps_S = jax.random.normal(jax.random.key(1), (NN, 64), jnp.float32)
    eps_Y = jax.random.normal(jax.random.key(2), (NN, 64), jnp.float32)
    noise = jax.random.uniform(jax.random.key(3), (NN, NN), jnp.float32)

    srcL = jnp.concatenate([src, jnp.arange(n)])
    dstL = jnp.concatenate([dst, jnp.arange(n)])

    def gcn_exact(x, W, b):
        xw = x @ W
        degf = jnp.zeros(n, xw.dtype).at[dstL].add(1.0)
        disf = jnp.where(degf > 0, degf ** -0.5, 0.0)
        normf = (disf[srcL] * disf[dstL])[:, None]
        out = jnp.zeros((n, W.shape[1]), xw.dtype).at[dstL].add(
            xw[srcL] * normf)
        return out + b

    def bn_exact(x):
        m = x.mean(axis=0)
        v = x.var(axis=0)
        return (x - m) / jnp.sqrt(v + 1e-5)

    h = jax.nn.relu(bn_exact(gcn_exact(X, uS_c1_W, uS_c1_b)))
    mu_S = gcn_exact(h, uS_mu_W, uS_mu_b)
    logvar_S = gcn_exact(h, uS_ls_W, uS_ls_b)
    xy = jnp.abs(jnp.concatenate([X, Y], axis=1))
    h2 = jax.nn.relu(bn_exact(gcn_exact(xy, uY_c1_W, uY_c1_b)))
    mu_Y = gcn_exact(h2, uY_mu_W, uY_mu_b)
    logvar_Y = gcn_exact(h2, uY_ls_W, uY_ls_b)
    u_S = _reparam(mu_S, logvar_S, eps_S)
    u_Y = _reparam(mu_Y, logvar_Y, eps_Y)

    dis = _dis_from_deg(_sc_degree(dst))

    def prop(x, w, b, mode, absx=False):
        c = w.shape[1]
        wp, bp = _pad128(w, b)
        t = _mmscale(x, wp, dis, absx=absx)
        acc = _sc_segsum(t, src, dst)
        out = _finish(acc, t, dis, bp, mode)
        return out if c == 128 else out[:, :c]

    feat = jnp.concatenate([u_S, u_Y], axis=1)
    au = _adj1(feat, noise)
    adj_prob, A_new = _adj2(au)
    l = adj_prob.reshape(-1)

    Xh = prop(feat, Xd1_W, Xd1_b, "bn_relu")
    X_new = prop(Xh, Xd2_W, Xd2_b, "bn")

    Yl = jnp.concatenate([u_Y, X], axis=1)
    Yh = prop(Yl, Yd1_W, Yd1_b, "bn_relu")
    Yo = prop(Yh, Yd2_W, Yd2_b, "bn")

    Sh = prop(u_S, Sd1_W, Sd1_b, "bn_relu")
    So = prop(Sh, Sd2_W, Sd2_b, "bn")

    Y_new, S_new, S_logits = _heads(Yo, So)

    return (mu_S, logvar_S, mu_Y, logvar_Y, u_S, u_Y, A_new, l, X_new,
            Y_new, S_new, S_logits)


# speedup vs baseline: 1.3302x; 1.0229x over previous
import functools

import jax
import jax.numpy as jnp
from jax import lax
from jax.experimental import pallas as pl
from jax.experimental.pallas import tpu as pltpu
from jax.experimental.pallas import tpu_sc as plsc

NN = 4096
HI = jax.lax.Precision.HIGHEST


def _mmcat_body(npair, pad, *refs):
    dis_ref = refs[2 * npair]
    o_ref = refs[2 * npair + 1]
    parts = [jnp.dot(refs[2 * i][...], refs[2 * i + 1][...],
                     preferred_element_type=jnp.float32)
             for i in range(npair)]
    if pad:
        parts.append(jnp.zeros((parts[0].shape[0], pad), jnp.float32))
    cat = parts[0] if len(parts) == 1 else jnp.concatenate(parts, axis=1)
    o_ref[...] = cat * dis_ref[...][:, None]


def _mmcat(pairs, dis):
    n = pairs[0][0].shape[0]
    pad = 128 - sum(w.shape[1] for _, w in pairs)
    args = [a for p in pairs for a in p]
    return pl.pallas_call(
        functools.partial(_mmcat_body, len(pairs), pad),
        out_shape=jax.ShapeDtypeStruct((n, 128), jnp.float32),
    )(*args, dis)


def _bn(p):
    m = jnp.mean(p, axis=0)
    v = jnp.mean((p - m) ** 2, axis=0)
    return (p - m) / jnp.sqrt(v + 1e-5)


def _finish_body(mode, acc_ref, t_ref, dis_ref, b_ref, o_ref):
    acc = acc_ref[0] + acc_ref[1]
    p = (acc + t_ref[...]) * dis_ref[...][:, None] + b_ref[...][None, :]
    if mode == "bn_relu":
        o_ref[...] = jnp.maximum(_bn(p), 0.0)
    elif mode == "bn":
        o_ref[...] = _bn(p)
    else:
        o_ref[...] = p


def _finish(acc, t, dis, b, mode):
    return pl.pallas_call(
        functools.partial(_finish_body, mode),
        out_shape=jax.ShapeDtypeStruct(t.shape, jnp.float32),
    )(acc, t, dis, b)


def _reparam_body(mu_ref, lv_ref, eps_ref, u_ref):
    u_ref[...] = eps_ref[...] * jnp.exp(0.5 * lv_ref[...]) + mu_ref[...]


def _reparam(mu, lv, eps):
    return pl.pallas_call(
        _reparam_body,
        out_shape=jax.ShapeDtypeStruct(mu.shape, jnp.float32),
    )(mu, lv, eps)


def _head_body(yo_ref, so_ref, yn_ref, sn_ref, sl_ref):
    yo = yo_ref[...]
    so = so_ref[...]
    yn_ref[...] = jax.nn.softmax(yo, axis=-1)
    sn_ref[...] = jax.nn.softmax(so, axis=-1)
    sl_ref[...] = jax.nn.sigmoid(so)


def _heads(yo, so):
    n = yo.shape[0]
    s2 = jax.ShapeDtypeStruct((n, 2), jnp.float32)
    return pl.pallas_call(_head_body, out_shape=(s2, s2, s2))(yo, so)


_AR = 512


def _adj1_body(fb_ref, ff_ref, nz_ref, au_ref):
    i = pl.program_id(0)
    raw = lax.dot_general(fb_ref[...], ff_ref[...],
                          (((1,), (1,)), ((), ())),
                          preferred_element_type=jnp.float32)
    rows = i * _AR + lax.broadcasted_iota(jnp.int32, raw.shape, 0)
    cols = lax.broadcasted_iota(jnp.int32, raw.shape, 1)
    upper = jnp.where(cols > rows, raw, 0.0)
    g = -jnp.log(-jnp.log(nz_ref[...] + 1e-9) + 1e-9)
    logits = upper + g
    m = jnp.max(logits, axis=-1, keepdims=True)
    e = jnp.exp(logits - m)
    au_ref[...] = e / jnp.sum(e, axis=-1, keepdims=True)


def _adj1(feat, noise):
    grid = NN // _AR
    return pl.pallas_call(
        _adj1_body,
        grid=(grid,),
        in_specs=[
            pl.BlockSpec((_AR, feat.shape[1]), lambda i: (i, 0)),
            pl.BlockSpec((NN, feat.shape[1]), lambda i: (0, 0)),
            pl.BlockSpec((_AR, NN), lambda i: (i, 0)),
        ],
        out_specs=pl.BlockSpec((_AR, NN), lambda i: (i, 0)),
        out_shape=jax.ShapeDtypeStruct((NN, NN), jnp.float32),
    )(feat, feat, noise)


def _adj2_body(a_ref, at_ref, prob_ref, an_ref):
    p = a_ref[...] + at_ref[...].T
    prob_ref[...] = p
    an_ref[...] = jnp.where(p > 0.5, 1.0, 0.0)


def _adj2(au):
    grid = NN // _AR
    s = jax.ShapeDtypeStruct((NN, NN), jnp.float32)
    return pl.pallas_call(
        _adj2_body,
        grid=(grid, grid),
        in_specs=[
            pl.BlockSpec((_AR, _AR), lambda i, j: (i, j)),
            pl.BlockSpec((_AR, _AR), lambda i, j: (j, i)),
        ],
        out_specs=[pl.BlockSpec((_AR, _AR), lambda i, j: (i, j)),
                   pl.BlockSpec((_AR, _AR), lambda i, j: (i, j))],
        out_shape=(s, s),
    )(au, au)


_NC = 2
_NS = 16
_NW = _NC * _NS
_EK = 128


def _sc_segsum(xs, srci, dsti):
    n, c = xs.shape
    e = srci.shape[0]
    per_w = e // _NW
    nch = per_w // _EK
    slab = n // _NS
    zslab = jnp.zeros((slab, c), jnp.float32)
    src2 = srci.reshape(e // _EK, _EK)
    dst2 = dsti.reshape(e // _EK, _EK)
    mesh = plsc.VectorSubcoreMesh(core_axis_name="c", subcore_axis_name="s")

    @functools.partial(
        pl.kernel, mesh=mesh,
        out_type=jax.ShapeDtypeStruct((_NC, n, c), jnp.float32),
        scratch_types=[
            pltpu.VMEM((nch, _EK), jnp.int32),
            pltpu.VMEM((nch, _EK), jnp.int32),
            pltpu.VMEM((_EK, c), jnp.float32),
            pltpu.VMEM((_EK, c), jnp.float32),
            pltpu.VMEM_SHARED((n, c), jnp.float32),
            pltpu.SemaphoreType.DMA,
            pltpu.SemaphoreType.DMA,
        ],
    )
    def k(xs_hbm, src_hbm, dst_hbm, z_hbm, out_hbm, sidx, didx, rows_a,
          rows_b, acc, sem_a, sem_b):
        cid = lax.axis_index("c")
        sid = lax.axis_index("s")
        wid = sid * _NC + cid
        pltpu.sync_copy(z_hbm, acc.at[pl.ds(sid * slab, slab)])
        pltpu.sync_copy(src_hbm.at[pl.ds(wid * nch, nch)], sidx)
        pltpu.sync_copy(dst_hbm.at[pl.ds(wid * nch, nch)], didx)
        plsc.subcore_barrier()
        bufs = (rows_a, rows_b)
        sems = (sem_a, sem_b)
        pltpu.async_copy(xs_hbm.at[sidx.at[0]], rows_a, sem_a)
        for j in range(nch):
            cur, sem = bufs[j % 2], sems[j % 2]
            if j + 1 < nch:
                pltpu.async_copy(xs_hbm.at[sidx.at[j + 1]],
                                 bufs[(j + 1) % 2], sems[(j + 1) % 2])
            pltpu.make_async_copy(xs_hbm.at[sidx.at[j]], cur, sem).wait()
            pltpu.sync_copy(cur, acc.at[didx.at[j]], add=True)
        plsc.subcore_barrier()
        pltpu.sync_copy(acc.at[pl.ds(sid * slab, slab)],
                        out_hbm.at[cid].at[pl.ds(sid * slab, slab)])

    return k(xs, src2, dst2, zslab)


def _sc_degree(dsti):
    n = NN
    e = dsti.shape[0]
    per_w = e // _NW
    nch = per_w // _EK
    slab = n // _NS
    ones = jnp.ones((_EK, 128), jnp.float32)
    zslab = jnp.zeros((slab, 128), jnp.float32)
    dst2 = dsti.reshape(e // _EK, _EK)
    mesh = plsc.VectorSubcoreMesh(core_axis_name="c", subcore_axis_name="s")

    @functools.partial(
        pl.kernel, mesh=mesh,
        out_type=jax.ShapeDtypeStruct((_NC, n, 128), jnp.float32),
        scratch_types=[
            pltpu.VMEM((nch, _EK), jnp.int32),
            pltpu.VMEM((_EK, 128), jnp.float32),
            pltpu.VMEM_SHARED((n, 128), jnp.float32),
        ],
    )
    def k(dst_hbm, ones_hbm, z_hbm, out_hbm, didx, onev, acc):
        cid = lax.axis_index("c")
        sid = lax.axis_index("s")
        wid = sid * _NC + cid
        pltpu.sync_copy(z_hbm, acc.at[pl.ds(sid * slab, slab)])
        pltpu.sync_copy(ones_hbm, onev)
        pltpu.sync_copy(dst_hbm.at[pl.ds(wid * nch, nch)], didx)
        plsc.subcore_barrier()
        for j in range(nch):
            pltpu.sync_copy(onev, acc.at[didx.at[j]], add=True)
        plsc.subcore_barrier()
        pltpu.sync_copy(acc.at[pl.ds(sid * slab, slab)],
                        out_hbm.at[cid].at[pl.ds(sid * slab, slab)])

    return k(dst2, ones, zslab)


def _dis_body(d_ref, o_ref):
    d = d_ref[0, :, 0] + d_ref[1, :, 0] + 1.0
    o_ref[...] = d ** -0.5


def _dis_from_deg(degout):
    return pl.pallas_call(
        _dis_body,
        out_shape=jax.ShapeDtypeStruct((NN,), jnp.float32),
    )(degout)


def kernel(X, edge_index, Y, idx, uS_c1_W, uS_c1_b, uS_mu_W, uS_mu_b,
           uS_ls_W, uS_ls_b, uY_c1_W, uY_c1_b, uY_mu_W, uY_mu_b, uY_ls_W,
           uY_ls_b, Xd1_W, Xd1_b, Xd2_W, Xd2_b, Yd1_W, Yd1_b, Yd2_W, Yd2_b,
           Sd1_W, Sd1_b, Sd2_W, Sd2_b):
    n = X.shape[0]
    src = edge_index[0]
    dst = edge_index[1]

    eps_S = jax.random.normal(jax.random.key(1), (NN, 64), jnp.float32)
    eps_Y = jax.random.normal(jax.random.key(2), (NN, 64), jnp.float32)
    noise = jax.random.uniform(jax.random.key(3), (NN, NN), jnp.float32)

    srcL = jnp.concatenate([src, jnp.arange(n)])
    dstL = jnp.concatenate([dst, jnp.arange(n)])

    def gcn_exact(x, W, b):
        xw = x @ W
        degf = jnp.zeros(n, xw.dtype).at[dstL].add(1.0)
        disf = jnp.where(degf > 0, degf ** -0.5, 0.0)
        normf = (disf[srcL] * disf[dstL])[:, None]
        out = jnp.zeros((n, W.shape[1]), xw.dtype).at[dstL].add(
            xw[srcL] * normf)
        return out + b

    def bn_exact(x):
        m = x.mean(axis=0)
        v = x.var(axis=0)
        return (x - m) / jnp.sqrt(v + 1e-5)

    h = jax.nn.relu(bn_exact(gcn_exact(X, uS_c1_W, uS_c1_b)))
    mu_S = gcn_exact(h, uS_mu_W, uS_mu_b)
    logvar_S = gcn_exact(h, uS_ls_W, uS_ls_b)
    xy = jnp.abs(jnp.concatenate([X, Y], axis=1))
    h2 = jax.nn.relu(bn_exact(gcn_exact(xy, uY_c1_W, uY_c1_b)))
    mu_Y = gcn_exact(h2, uY_mu_W, uY_mu_b)
    logvar_Y = gcn_exact(h2, uY_ls_W, uY_ls_b)
    u_S = _reparam(mu_S, logvar_S, eps_S)
    u_Y = _reparam(mu_Y, logvar_Y, eps_Y)

    dis = _dis_from_deg(_sc_degree(dst))

    feat = jnp.concatenate([u_S, u_Y], axis=1)
    au = _adj1(feat, noise)
    adj_prob, A_new = _adj2(au)
    l = adj_prob.reshape(-1)

    Yl = jnp.concatenate([u_Y, X], axis=1)
    t1a = _mmcat([(feat, Xd1_W), (Yl, Yd1_W)], dis)
    b1a = jnp.concatenate([Xd1_b, Yd1_b])
    s1a = _finish(_sc_segsum(t1a, src, dst), t1a, dis, b1a, "bn_relu")
    Xh, Yh = s1a[:, :64], s1a[:, 64:128]
    t1b = _mmcat([(u_S, Sd1_W)], dis)
    b1b = jnp.concatenate([Sd1_b, jnp.zeros(64, jnp.float32)])
    s1b = _finish(_sc_segsum(t1b, src, dst), t1b, dis, b1b, "bn_relu")
    Sh = s1b[:, :64]

    t2a = _mmcat([(Xh, Xd2_W)], dis)
    s2a = _finish(_sc_segsum(t2a, src, dst), t2a, dis, Xd2_b, "bn")
    X_new = s2a
    t2b = _mmcat([(Yh, Yd2_W), (Sh, Sd2_W)], dis)
    b2b = jnp.concatenate([Yd2_b, Sd2_b, jnp.zeros(124, jnp.float32)])
    s2b = _finish(_sc_segsum(t2b, src, dst), t2b, dis, b2b, "bn")
    Yo = s2b[:, :2]
    So = s2b[:, 2:4]

    Y_new, S_new, S_logits = _heads(Yo, So)

    return (mu_S, logvar_S, mu_Y, logvar_Y, u_S, u_Y, A_new, l, X_new,
            Y_new, S_new, S_logits)


# speedup vs baseline: 1.3323x; 1.0015x over previous
import functools

import jax
import jax.numpy as jnp
from jax import lax
from jax.experimental import pallas as pl
from jax.experimental.pallas import tpu as pltpu
from jax.experimental.pallas import tpu_sc as plsc

NN = 4096
HI = jax.lax.Precision.HIGHEST


def _mmcat_body(npair, pad, *refs):
    dis_ref = refs[2 * npair]
    o_ref = refs[2 * npair + 1]
    parts = [jnp.dot(refs[2 * i][...], refs[2 * i + 1][...],
                     preferred_element_type=jnp.float32)
             for i in range(npair)]
    if pad:
        parts.append(jnp.zeros((parts[0].shape[0], pad), jnp.float32))
    cat = parts[0] if len(parts) == 1 else jnp.concatenate(parts, axis=1)
    o_ref[...] = cat * dis_ref[...][:, None]


def _mmcat(pairs, dis):
    n = pairs[0][0].shape[0]
    pad = 128 - sum(w.shape[1] for _, w in pairs)
    args = [a for p in pairs for a in p]
    return pl.pallas_call(
        functools.partial(_mmcat_body, len(pairs), pad),
        out_shape=jax.ShapeDtypeStruct((n, 128), jnp.float32),
    )(*args, dis)


def _bn(p):
    m = jnp.mean(p, axis=0)
    v = jnp.mean((p - m) ** 2, axis=0)
    return (p - m) / jnp.sqrt(v + 1e-5)


def _finish_body(mode, acc_ref, t_ref, dis_ref, b_ref, o_ref):
    acc = acc_ref[0] + acc_ref[1]
    p = (acc + t_ref[...]) * dis_ref[...][:, None] + b_ref[...][None, :]
    if mode == "bn_relu":
        o_ref[...] = jnp.maximum(_bn(p), 0.0)
    elif mode == "bn":
        o_ref[...] = _bn(p)
    else:
        o_ref[...] = p


def _finish(acc, t, dis, b, mode):
    return pl.pallas_call(
        functools.partial(_finish_body, mode),
        out_shape=jax.ShapeDtypeStruct(t.shape, jnp.float32),
    )(acc, t, dis, b)


def _reparam_body(mu_ref, lv_ref, eps_ref, u_ref):
    u_ref[...] = eps_ref[...] * jnp.exp(0.5 * lv_ref[...]) + mu_ref[...]


def _reparam(mu, lv, eps):
    return pl.pallas_call(
        _reparam_body,
        out_shape=jax.ShapeDtypeStruct(mu.shape, jnp.float32),
    )(mu, lv, eps)


def _head_body(yo_ref, so_ref, yn_ref, sn_ref, sl_ref):
    yo = yo_ref[...]
    so = so_ref[...]
    yn_ref[...] = jax.nn.softmax(yo, axis=-1)
    sn_ref[...] = jax.nn.softmax(so, axis=-1)
    sl_ref[...] = jax.nn.sigmoid(so)


def _heads(yo, so):
    n = yo.shape[0]
    s2 = jax.ShapeDtypeStruct((n, 2), jnp.float32)
    return pl.pallas_call(_head_body, out_shape=(s2, s2, s2))(yo, so)


_AR = 512


def _adj1_body(fb_ref, ff_ref, nz_ref, au_ref):
    i = pl.program_id(0)
    raw = lax.dot_general(fb_ref[...], ff_ref[...],
                          (((1,), (1,)), ((), ())),
                          preferred_element_type=jnp.float32)
    rows = i * _AR + lax.broadcasted_iota(jnp.int32, raw.shape, 0)
    cols = lax.broadcasted_iota(jnp.int32, raw.shape, 1)
    upper = jnp.where(cols > rows, raw, 0.0)
    g = -jnp.log(-jnp.log(nz_ref[...] + 1e-9) + 1e-9)
    logits = upper + g
    m = jnp.max(logits, axis=-1, keepdims=True)
    e = jnp.exp(logits - m)
    au_ref[...] = e / jnp.sum(e, axis=-1, keepdims=True)


def _adj1(feat, noise):
    grid = NN // _AR
    return pl.pallas_call(
        _adj1_body,
        grid=(grid,),
        in_specs=[
            pl.BlockSpec((_AR, feat.shape[1]), lambda i: (i, 0)),
            pl.BlockSpec((NN, feat.shape[1]), lambda i: (0, 0)),
            pl.BlockSpec((_AR, NN), lambda i: (i, 0)),
        ],
        out_specs=pl.BlockSpec((_AR, NN), lambda i: (i, 0)),
        out_shape=jax.ShapeDtypeStruct((NN, NN), jnp.float32),
    )(feat, feat, noise)


def _adj2_body(a_ref, at_ref, prob_ref, an_ref):
    p = a_ref[...] + at_ref[...].T
    prob_ref[...] = p
    an_ref[...] = jnp.where(p > 0.5, 1.0, 0.0)


def _adj2(au):
    grid = NN // _AR
    s = jax.ShapeDtypeStruct((NN, NN), jnp.float32)
    return pl.pallas_call(
        _adj2_body,
        grid=(grid, grid),
        in_specs=[
            pl.BlockSpec((_AR, _AR), lambda i, j: (i, j)),
            pl.BlockSpec((_AR, _AR), lambda i, j: (j, i)),
        ],
        out_specs=[pl.BlockSpec((_AR, _AR), lambda i, j: (i, j)),
                   pl.BlockSpec((_AR, _AR), lambda i, j: (i, j))],
        out_shape=(s, s),
    )(au, au)


_NC = 2
_NS = 16
_NW = _NC * _NS
_EK = 128


def _sc_segsum2(xs1, xs2, srci, dsti):
    n, c = xs1.shape
    e = srci.shape[0]
    per_w = e // _NW
    nch = per_w // _EK
    slab = n // _NS
    zslab = jnp.zeros((slab, c), jnp.float32)
    src2 = srci.reshape(e // _EK, _EK)
    dst2 = dsti.reshape(e // _EK, _EK)
    mesh = plsc.VectorSubcoreMesh(core_axis_name="c", subcore_axis_name="s")

    @functools.partial(
        pl.kernel, mesh=mesh,
        out_type=jax.ShapeDtypeStruct((2 * _NC, n, c), jnp.float32),
        scratch_types=[
            pltpu.VMEM((nch, _EK), jnp.int32),
            pltpu.VMEM((nch, _EK), jnp.int32),
            pltpu.VMEM((_EK, c), jnp.float32),
            pltpu.VMEM((_EK, c), jnp.float32),
            pltpu.VMEM_SHARED((n, c), jnp.float32),
            pltpu.SemaphoreType.DMA,
            pltpu.SemaphoreType.DMA,
        ],
    )
    def k(x1_hbm, x2_hbm, src_hbm, dst_hbm, z_hbm, out_hbm, sidx, didx,
          rows_a, rows_b, acc, sem_a, sem_b):
        cid = lax.axis_index("c")
        sid = lax.axis_index("s")
        wid = sid * _NC + cid
        pltpu.sync_copy(src_hbm.at[pl.ds(wid * nch, nch)], sidx)
        pltpu.sync_copy(dst_hbm.at[pl.ds(wid * nch, nch)], didx)
        bufs = (rows_a, rows_b)
        sems = (sem_a, sem_b)
        for h, x_hbm in enumerate((x1_hbm, x2_hbm)):
            pltpu.sync_copy(z_hbm, acc.at[pl.ds(sid * slab, slab)])
            plsc.subcore_barrier()
            pltpu.async_copy(x_hbm.at[sidx.at[0]], rows_a, sem_a)
            for j in range(nch):
                cur, sem = bufs[j % 2], sems[j % 2]
                if j + 1 < nch:
                    pltpu.async_copy(x_hbm.at[sidx.at[j + 1]],
                                     bufs[(j + 1) % 2], sems[(j + 1) % 2])
                pltpu.make_async_copy(x_hbm.at[sidx.at[j]], cur, sem).wait()
                pltpu.sync_copy(cur, acc.at[didx.at[j]], add=True)
            plsc.subcore_barrier()
            pltpu.sync_copy(acc.at[pl.ds(sid * slab, slab)],
                            out_hbm.at[2 * h + cid].at[pl.ds(sid * slab,
                                                             slab)])

    out = k(xs1, xs2, src2, dst2, zslab)
    return out[0:2], out[2:4]


def kernel(X, edge_index, Y, idx, uS_c1_W, uS_c1_b, uS_mu_W, uS_mu_b,
           uS_ls_W, uS_ls_b, uY_c1_W, uY_c1_b, uY_mu_W, uY_mu_b, uY_ls_W,
           uY_ls_b, Xd1_W, Xd1_b, Xd2_W, Xd2_b, Yd1_W, Yd1_b, Yd2_W, Yd2_b,
           Sd1_W, Sd1_b, Sd2_W, Sd2_b):
    n = X.shape[0]
    src = edge_index[0]
    dst = edge_index[1]

    eps_S = jax.random.normal(jax.random.key(1), (NN, 64), jnp.float32)
    eps_Y = jax.random.normal(jax.random.key(2), (NN, 64), jnp.float32)
    noise = jax.random.uniform(jax.random.key(3), (NN, NN), jnp.float32)

    srcL = jnp.concatenate([src, jnp.arange(n)])
    dstL = jnp.concatenate([dst, jnp.arange(n)])

    def gcn_exact(x, W, b):
        xw = x @ W
        degf = jnp.zeros(n, xw.dtype).at[dstL].add(1.0)
        disf = jnp.where(degf > 0, degf ** -0.5, 0.0)
        normf = (disf[srcL] * disf[dstL])[:, None]
        out = jnp.zeros((n, W.shape[1]), xw.dtype).at[dstL].add(
            xw[srcL] * normf)
        return out + b

    def bn_exact(x):
        m = x.mean(axis=0)
        v = x.var(axis=0)
        return (x - m) / jnp.sqrt(v + 1e-5)

    h = jax.nn.relu(bn_exact(gcn_exact(X, uS_c1_W, uS_c1_b)))
    mu_S = gcn_exact(h, uS_mu_W, uS_mu_b)
    logvar_S = gcn_exact(h, uS_ls_W, uS_ls_b)
    xy = jnp.abs(jnp.concatenate([X, Y], axis=1))
    h2 = jax.nn.relu(bn_exact(gcn_exact(xy, uY_c1_W, uY_c1_b)))
    mu_Y = gcn_exact(h2, uY_mu_W, uY_mu_b)
    logvar_Y = gcn_exact(h2, uY_ls_W, uY_ls_b)
    u_S = _reparam(mu_S, logvar_S, eps_S)
    u_Y = _reparam(mu_Y, logvar_Y, eps_Y)

    degd = jnp.zeros(n, jnp.float32).at[dstL].add(1.0)
    dis = jnp.where(degd > 0, degd ** -0.5, 0.0)

    feat = jnp.concatenate([u_S, u_Y], axis=1)
    au = _adj1(feat, noise)
    adj_prob, A_new = _adj2(au)
    l = adj_prob.reshape(-1)

    Yl = jnp.concatenate([u_Y, X], axis=1)
    t1a = _mmcat([(feat, Xd1_W), (Yl, Yd1_W)], dis)
    b1a = jnp.concatenate([Xd1_b, Yd1_b])
    t1b = _mmcat([(u_S, Sd1_W)], dis)
    b1b = jnp.concatenate([Sd1_b, jnp.zeros(64, jnp.float32)])
    o1a, o1b = _sc_segsum2(t1a, t1b, src, dst)
    s1a = _finish(o1a, t1a, dis, b1a, "bn_relu")
    Xh, Yh = s1a[:, :64], s1a[:, 64:128]
    s1b = _finish(o1b, t1b, dis, b1b, "bn_relu")
    Sh = s1b[:, :64]

    t2a = _mmcat([(Xh, Xd2_W)], dis)
    t2b = _mmcat([(Yh, Yd2_W), (Sh, Sd2_W)], dis)
    b2b = jnp.concatenate([Yd2_b, Sd2_b, jnp.zeros(124, jnp.float32)])
    o2a, o2b = _sc_segsum2(t2a, t2b, src, dst)
    X_new = _finish(o2a, t2a, dis, Xd2_b, "bn")
    s2b = _finish(o2b, t2b, dis, b2b, "bn")
    Yo = s2b[:, :2]
    So = s2b[:, 2:4]

    Y_new, S_new, S_logits = _heads(Yo, So)

    return (mu_S, logvar_S, mu_Y, logvar_Y, u_S, u_Y, A_new, l, X_new,
            Y_new, S_new, S_logits)
